# baseline (speedup 1.0000x reference)
import jax
import jax.numpy as jnp
from jax import lax
from jax.experimental import pallas as pl
from jax.experimental.pallas import tpu as pltpu
from jax.experimental.pallas import tpu_sc as plsc

D = 4096
B = 8192
NC = 2
NS = 16
NW = NC * NS
ROWS_PER_W = B // NW
K = 8
CHUNKS = ROWS_PER_W // K
UNROLL = 8
MASK = -65536
NBUF = 3


def _emb_body(ids_hbm, wte_hbm, out_hbm, ids_v, idx_u, rows_0, rows_1,
              rows_2, sem_0, sem_1, sem_2):
    bufs = (rows_0, rows_1, rows_2)
    sems = (sem_0, sem_1, sem_2)
    wid = lax.axis_index("s") * NC + lax.axis_index("c")
    base = wid * ROWS_PER_W
    pltpu.sync_copy(ids_hbm.at[pl.ds(base, ROWS_PER_W)],
                    ids_v.at[pl.ds(0, ROWS_PER_W)])
    wte32 = wte_hbm.bitcast(jnp.int32)
    out32 = out_hbm.bitcast(jnp.int32)

    def mk_idx(i, carry):
        idx_u[pl.ds(i * 16, 16)] = ids_v[pl.ds(i * 16, 16)] >> 1
        return carry

    lax.fori_loop(0, ROWS_PER_W // 16, mk_idx, 0)

    def gather(c, buf, sem):
        return pltpu.async_copy(wte32.at[idx_u.at[pl.ds(c * K, K)]], buf, sem)

    def step(c, buf, sem, nxt_buf, nxt_sem, prefetch=True):
        pltpu.make_async_copy(
            wte32.at[idx_u.at[pl.ds(0, K)]], buf, sem).wait()
        if prefetch:
            gather(c + 2, nxt_buf, nxt_sem)

        gvec = ids_v[pl.ds(c * K, 16)]
        for r in range(K):
            v_r = gvec[r]
            shift = 16 - ((v_r & 1) << 4)

            @plsc.parallel_loop(0, D // 16, 1, unroll=UNROLL)
            def _(j, r=r, shift=shift):
                sl = pl.ds(j * 16, 16)
                buf[r, sl] = (buf[r, sl] << shift) & jnp.int32(MASK)
        pltpu.sync_copy(buf, out32.at[pl.ds(base + c * K, K), :])

    gather(0, bufs[0], sems[0])
    gather(1, bufs[1], sems[1])

    def body(t, carry):
        for s in range(NBUF):
            c = NBUF * t + s
            step(c, bufs[s], sems[s],
                 bufs[(s + 2) % NBUF], sems[(s + 2) % NBUF])
        return carry

    lax.fori_loop(0, CHUNKS // NBUF, body, 0)
    for c in range(CHUNKS - CHUNKS % NBUF, CHUNKS):
        s = c % NBUF
        step(c, bufs[s], sems[s], bufs[(s + 2) % NBUF], sems[(s + 2) % NBUF],
             prefetch=False)


@jax.jit
def _emb(ids_flat, wte):
    mesh = plsc.VectorSubcoreMesh(core_axis_name="c", subcore_axis_name="s")
    f = pl.kernel(
        _emb_body,
        mesh=mesh,
        compiler_params=pltpu.CompilerParams(needs_layout_passes=False),
        out_type=jax.ShapeDtypeStruct((B, D), jnp.float32),
        scratch_types=[
            pltpu.VMEM((ROWS_PER_W + 16,), jnp.int32),
            pltpu.VMEM((ROWS_PER_W,), jnp.int32),
            pltpu.VMEM((K, D), jnp.int32),
            pltpu.VMEM((K, D), jnp.int32),
            pltpu.VMEM((K, D), jnp.int32),
            pltpu.SemaphoreType.DMA,
            pltpu.SemaphoreType.DMA,
            pltpu.SemaphoreType.DMA,
        ],
    )
    return f(ids_flat, wte)


def kernel(input_ids, wte):
    out = _emb(input_ids.reshape(-1), wte)
    return out.reshape(input_ids.shape[0], input_ids.shape[1], D)


# speedup vs baseline: 1.0248x; 1.0248x over previous
import jax
import jax.numpy as jnp
from jax import lax
from jax.experimental import pallas as pl
from jax.experimental.pallas import tpu as pltpu
from jax.experimental.pallas import tpu_sc as plsc

D = 4096
BATCH = 4
SEQ = 2048
B = BATCH * SEQ
NC = 2
NS = 16
NW = NC * NS
ROWS_PER_W = B // NW
K = 8
CHUNKS = ROWS_PER_W // K
UNROLL = 8
MASK = -65536


def _emb_body(ids_hbm, wte_hbm, out_hbm, ids_v, idx_u, rows_a, rows_b,
              sem_a, sem_b):
    wid = lax.axis_index("s") * NC + lax.axis_index("c")
    base = wid * ROWS_PER_W
    pltpu.sync_copy(ids_hbm.at[pl.ds(base, ROWS_PER_W)], ids_v)
    wte32 = wte_hbm.bitcast(jnp.int32)
    out32 = out_hbm.bitcast(jnp.int32)

    def mk_idx(i, carry):
        idx_u[pl.ds(i * 16, 16)] = ids_v[pl.ds(i * 16, 16)] >> 1
        return carry

    lax.fori_loop(0, ROWS_PER_W // 16, mk_idx, 0)

    def gather(c, buf, sem):
        return pltpu.async_copy(wte32.at[idx_u.at[pl.ds(c * K, K)]], buf, sem)

    def convert_and_store(c, rows_v, gvec, lane0):
        for r in range(K):
            v_r = gvec[lane0 + r]
            shift = 16 - ((v_r & 1) << 4)

            @plsc.parallel_loop(0, D // 16, 1, unroll=UNROLL)
            def _(j, r=r, shift=shift):
                sl = pl.ds(j * 16, 16)
                rows_v[r, sl] = (rows_v[r, sl] << shift) & jnp.int32(MASK)
        pltpu.sync_copy(rows_v, out32.at[pl.ds(base + c * K, K), :])

    gather(0, rows_a, sem_a)

    def body(t, carry):
        ca = 2 * t
        cb = 2 * t + 1
        gather(cb, rows_b, sem_b)
        gvec = ids_v[pl.ds(t * 16, 16)]
        pltpu.make_async_copy(
            wte32.at[idx_u.at[pl.ds(0, K)]], rows_a, sem_a).wait()
        convert_and_store(ca, rows_a, gvec, 0)

        @pl.when(t < CHUNKS // 2 - 1)
        def _():
            gather(ca + 2, rows_a, sem_a)

        pltpu.make_async_copy(
            wte32.at[idx_u.at[pl.ds(0, K)]], rows_b, sem_b).wait()
        convert_and_store(cb, rows_b, gvec, 8)
        return carry

    lax.fori_loop(0, CHUNKS // 2, body, 0)


@jax.jit
def _emb(ids_flat, wte):
    mesh = plsc.VectorSubcoreMesh(core_axis_name="c", subcore_axis_name="s")
    f = pl.kernel(
        _emb_body,
        mesh=mesh,
        compiler_params=pltpu.CompilerParams(needs_layout_passes=False),
        out_type=jax.ShapeDtypeStruct((B, D), jnp.float32),
        scratch_types=[
            pltpu.VMEM((ROWS_PER_W,), jnp.int32),
            pltpu.VMEM((ROWS_PER_W,), jnp.int32),
            pltpu.VMEM((K, D), jnp.int32),
            pltpu.VMEM((K, D), jnp.int32),
            pltpu.SemaphoreType.DMA,
            pltpu.SemaphoreType.DMA,
        ],
    )
    return f(ids_flat, wte)


def kernel(input_ids, wte):
    out = _emb(input_ids.reshape(-1), wte)
    return out.reshape(input_ids.shape[0], input_ids.shape[1], D)
